# baseline (speedup 1.0000x reference)
import functools

import jax
import jax.numpy as jnp
from jax import lax
from jax.experimental import pallas as pl
from jax.experimental.pallas import tpu as pltpu
from jax.experimental.pallas import tpu_sc as plsc

N, L, D = 32, 1024, 256
B = N * L
LANES = 16
NUM_WORKERS = 32
B_PER_W = B // NUM_WORKERS
CHUNK = 128
N_CHUNKS = B_PER_W // CHUNK


def _make_lookup():
  mesh = plsc.VectorSubcoreMesh(core_axis_name="c", subcore_axis_name="s")

  @functools.partial(
      pl.kernel,
      mesh=mesh,
      compiler_params=pltpu.CompilerParams(
          use_tc_tiling_on_sc=False, needs_layout_passes=False
      ),
      out_type=jax.ShapeDtypeStruct((B, D), jnp.float32),
      scratch_types=[
          pltpu.VMEM((B_PER_W,), jnp.int32),
          pltpu.VMEM((B_PER_W,), jnp.int32),
          pltpu.VMEM((8 * D,), jnp.float32),
          pltpu.VMEM((CHUNK, D), jnp.float32),
          pltpu.VMEM((CHUNK, D), jnp.float32),
          pltpu.SemaphoreType.DMA,
          pltpu.SemaphoreType.DMA,
      ],
  )
  def lookup(aa_hbm, mask_hbm, table_hbm, out_hbm, idx_v, mask_v, tab_v,
             rows_a, rows_b, ss_a, ss_b):
    wid = lax.axis_index("s") * 2 + lax.axis_index("c")
    base = wid * B_PER_W
    pltpu.sync_copy(table_hbm, tab_v)
    pltpu.sync_copy(aa_hbm.at[pl.ds(base, B_PER_W)], idx_v)
    pltpu.sync_copy(mask_hbm.at[pl.ds(base, B_PER_W)], mask_v)
    iota = lax.iota(jnp.int32, LANES)
    for j in range(B_PER_W // LANES):
      sl = pl.ds(j * LANES, LANES)
      a = idx_v[sl]
      m = mask_v[sl]
      idx_v[sl] = jnp.where(m == 0, jnp.int32(7), a) * D

    def expand(c, buf):
      def group_body(g, _):
        k256 = idx_v[pl.ds(c * CHUNK + g * LANES, LANES)]
        rvec = g * LANES + iota

        def col_body(jj, _):
          for t in range(16):
            j = jj * 16 + t
            vals = plsc.load_gather(tab_v, [k256 + j])
            plsc.store_scatter(buf, [rvec, jnp.full((LANES,), j, jnp.int32)],
                               vals)
          return 0

        lax.fori_loop(0, D // 16, col_body, 0)
        return 0

      lax.fori_loop(0, CHUNK // LANES, group_body, 0)

    rows = (rows_a, rows_b)
    ss = (ss_a, ss_b)
    s = [None] * N_CHUNKS
    for c in range(N_CHUNKS):
      b = c % 2
      if c >= 2:
        s[c - 2].wait()
      expand(c, rows[b])
      s[c] = pltpu.async_copy(
          rows[b], out_hbm.at[pl.ds(base + c * CHUNK, CHUNK)], ss[b]
      )
    s[N_CHUNKS - 2].wait()
    s[N_CHUNKS - 1].wait()

  return lookup


_lookup = _make_lookup()


def kernel(aa, res_nb, chain_nb, pos_atoms, mask_atoms, fragment_type, emb_table):
  aa_flat = aa.reshape(B).astype(jnp.int32)
  mask_flat = mask_atoms[:, :, 0].reshape(B).astype(jnp.int32)
  table_ext = jnp.concatenate(
      [emb_table.astype(jnp.float32), jnp.zeros((1, D), jnp.float32)], axis=0
  )
  out = _lookup(aa_flat, mask_flat, table_ext.reshape(8 * D))
  return out.reshape(N, L, D)


# speedup vs baseline: 6.9584x; 6.9584x over previous
import functools

import jax
import jax.numpy as jnp
from jax import lax
from jax.experimental import pallas as pl
from jax.experimental.pallas import tpu as pltpu
from jax.experimental.pallas import tpu_sc as plsc

N, L, D = 32, 1024, 256
B = N * L
LANES = 16
NUM_WORKERS = 32
B_PER_W = B // NUM_WORKERS
CHUNK = 128
N_CHUNKS = B_PER_W // CHUNK
REPS = 16


def _make_lookup():
  mesh = plsc.VectorSubcoreMesh(core_axis_name="c", subcore_axis_name="s")

  @functools.partial(
      pl.kernel,
      mesh=mesh,
      out_type=jax.ShapeDtypeStruct((B, D), jnp.float32),
      scratch_types=[
          pltpu.VMEM((B_PER_W,), jnp.int32),
          pltpu.VMEM((B_PER_W,), jnp.int32),
          pltpu.VMEM((CHUNK, D), jnp.float32),
          pltpu.VMEM((CHUNK, D), jnp.float32),
          pltpu.VMEM((CHUNK, D), jnp.float32),
          pltpu.SemaphoreType.DMA,
          pltpu.SemaphoreType.DMA,
          pltpu.SemaphoreType.DMA,
          pltpu.SemaphoreType.DMA,
          pltpu.SemaphoreType.DMA,
          pltpu.SemaphoreType.DMA,
      ],
  )
  def lookup(aa_hbm, mask_hbm, table_hbm, out_hbm, idx_v, mask_v, rows_a,
             rows_b, rows_c, sg_a, sg_b, sg_c, ss_a, ss_b, ss_c):
    wid = lax.axis_index("s") * 2 + lax.axis_index("c")
    base = wid * B_PER_W
    pltpu.sync_copy(aa_hbm.at[pl.ds(base, B_PER_W)], idx_v)
    pltpu.sync_copy(mask_hbm.at[pl.ds(base, B_PER_W)], mask_v)
    tab_base = wid * (8 * REPS)
    for j in range(B_PER_W // LANES):
      sl = pl.ds(j * LANES, LANES)
      a = idx_v[sl]
      m = mask_v[sl]
      idx_v[sl] = jnp.where(m == 0, jnp.int32(7), a) + (
          tab_base + (j % REPS) * 8
      )
    rows = (rows_a, rows_b, rows_c)
    sg = (sg_a, sg_b, sg_c)
    ss = (ss_a, ss_b, ss_c)
    nbuf = len(rows)
    g = [None] * N_CHUNKS
    s = [None] * N_CHUNKS
    for c in range(N_CHUNKS):
      b = c % nbuf
      if c >= nbuf:
        s[c - nbuf].wait()
      g[c] = pltpu.async_copy(
          table_hbm.at[idx_v.at[pl.ds(c * CHUNK, CHUNK)]], rows[b], sg[b]
      )
      if c >= 1:
        p = c - 1
        g[p].wait()
        s[p] = pltpu.async_copy(
            rows[p % nbuf], out_hbm.at[pl.ds(base + p * CHUNK, CHUNK)],
            ss[p % nbuf]
        )
    last = N_CHUNKS - 1
    g[last].wait()
    s[last] = pltpu.async_copy(
        rows[last % nbuf], out_hbm.at[pl.ds(base + last * CHUNK, CHUNK)],
        ss[last % nbuf]
    )
    for c in range(max(0, N_CHUNKS - nbuf), N_CHUNKS):
      s[c].wait()

  return lookup


_lookup = _make_lookup()


def kernel(aa, res_nb, chain_nb, pos_atoms, mask_atoms, fragment_type, emb_table):
  aa_flat = aa.reshape(B).astype(jnp.int32)
  mask_flat = mask_atoms[:, :, 0].reshape(B).astype(jnp.int32)
  table_ext = jnp.concatenate(
      [emb_table.astype(jnp.float32), jnp.zeros((1, D), jnp.float32)], axis=0
  )
  table_rep = jnp.tile(table_ext, (NUM_WORKERS * REPS, 1))
  out = _lookup(aa_flat, mask_flat, table_rep)
  return out.reshape(N, L, D)
